# baseline (speedup 1.0000x reference)
import functools

import jax
import jax.numpy as jnp
from jax.experimental import pallas as pl
from jax.experimental.pallas import tpu as pltpu

_INF = 1e30
_BIG_LABEL = 1e9
_K = 5


def _extract_top5(cur, labf):
    vals, labs = [], []
    for _ in range(_K):
        m = jnp.min(cur, axis=1, keepdims=True)
        eq = cur == m
        l = jnp.min(jnp.where(eq, labf, _BIG_LABEL), axis=1, keepdims=True)
        vals.append(m)
        labs.append(l)
        cur = jnp.where(eq, _INF, cur)
    return jnp.concatenate(vals, axis=1), jnp.concatenate(labs, axis=1)


def _knn_body(n_keys, blk_rows, x_ref, data_ref, labels_ref, out_ref,
              vals_ref, labs_ref):
    i = pl.program_id(0)
    nb = pl.num_programs(0)
    q = x_ref.shape[0]

    @pl.when(i == 0)
    def _init():
        vals_ref[...] = jnp.full(vals_ref.shape, _INF, jnp.float32)
        labs_ref[...] = jnp.zeros(labs_ref.shape, jnp.float32)

    xq = x_ref[...]
    blk = data_ref[...]
    s = jax.lax.dot_general(xq, blk, (((1,), (1,)), ((), ())),
                            preferred_element_type=jnp.float32)
    ones_row = jnp.ones((1, xq.shape[1]), jnp.float32)
    k2 = jax.lax.dot_general(ones_row, blk * blk, (((1,), (1,)), ((), ())),
                             preferred_element_type=jnp.float32)
    t = k2 - 2.0 * s

    col = jax.lax.broadcasted_iota(jnp.int32, (1, blk_rows), 1) + i * blk_rows
    t = jnp.where(col < n_keys, t, _INF)

    labf = labels_ref[...].astype(jnp.float32).reshape(1, blk_rows)
    bv, bl = _extract_top5(t, labf)

    pad_v = jnp.full((q, 3), _INF, jnp.float32)
    pad_l = jnp.zeros((q, 3), jnp.float32)
    cv = jnp.concatenate([vals_ref[:, 0:8], bv, pad_v], axis=1)
    cl = jnp.concatenate([labs_ref[:, 0:8], bl, pad_l], axis=1)
    mv, ml = _extract_top5(cv, cl)
    vals_ref[:, 0:8] = jnp.concatenate([mv, pad_v], axis=1)
    labs_ref[:, 0:8] = jnp.concatenate([ml, pad_l], axis=1)

    @pl.when(i == nb - 1)
    def _finish():
        q2 = jnp.sum(xq * xq, axis=1, keepdims=True)
        d2 = jnp.maximum(vals_ref[:, 0:8] + q2, 0.0)
        dist = jnp.sqrt(d2)
        w = 1.0 / jnp.maximum(dist, 1e-12)
        lab = labs_ref[:, 0:8]
        g_cols = []
        for k in range(8):
            same = lab[:, k:k + 1] == lab
            g_cols.append(jnp.sum(jnp.where(same, w, 0.0), axis=1,
                                  keepdims=True))
        g = jnp.concatenate(g_cols, axis=1)
        gm = jnp.max(g, axis=1, keepdims=True)
        pred = jnp.min(jnp.where(g == gm, lab, _BIG_LABEL), axis=1,
                       keepdims=True)
        cls = jax.lax.broadcasted_iota(jnp.int32, out_ref.shape, 1)
        out_ref[...] = (cls.astype(jnp.float32) == pred).astype(jnp.float32)


def kernel(x, data, labels, classes_mask):
    del classes_mask
    n_keys = data.shape[0]
    q = x.shape[0]
    n_classes = 1000
    blk_rows = 2048
    nb = pl.cdiv(n_keys, blk_rows)

    body = functools.partial(_knn_body, n_keys, blk_rows)
    return pl.pallas_call(
        body,
        grid=(nb,),
        in_specs=[
            pl.BlockSpec((q, data.shape[1]), lambda i: (0, 0)),
            pl.BlockSpec((blk_rows, data.shape[1]), lambda i: (i, 0)),
            pl.BlockSpec((blk_rows,), lambda i: (i,)),
        ],
        out_specs=pl.BlockSpec((q, n_classes), lambda i: (0, 0)),
        out_shape=jax.ShapeDtypeStruct((q, n_classes), jnp.float32),
        scratch_shapes=[
            pltpu.VMEM((q, 128), jnp.float32),
            pltpu.VMEM((q, 128), jnp.float32),
        ],
        compiler_params=pltpu.CompilerParams(
            dimension_semantics=("arbitrary",),
        ),
    )(x, data, labels)


# speedup vs baseline: 1.6567x; 1.6567x over previous
import functools

import jax
import jax.numpy as jnp
from jax.experimental import pallas as pl
from jax.experimental.pallas import tpu as pltpu

_INF = 1e30
_BIG_LABEL = 1e9
_K = 5


def _extract_top5(cur, labf):
    q = cur.shape[0]
    vals, labs = [], []
    for _ in range(_K):
        m = jnp.min(cur, axis=1, keepdims=True)
        eq = cur == m
        l = jnp.min(jnp.where(eq, labf, _BIG_LABEL), axis=1, keepdims=True)
        vals.append(m)
        labs.append(l)
        cur = jnp.where(eq, _INF, cur)
    vals.append(jnp.full((q, 3), _INF, jnp.float32))
    labs.append(jnp.zeros((q, 3), jnp.float32))
    return jnp.concatenate(vals, axis=1), jnp.concatenate(labs, axis=1)


def _knn_body(n_keys, blk_rows, n_panels, x_ref, data_ref, labels_ref,
              out_ref, cval_ref, clab_ref):
    i = pl.program_id(0)
    nb = pl.num_programs(0)
    q = x_ref.shape[0]
    pw = blk_rows // n_panels

    xq = x_ref[...]
    blk = data_ref[...]
    s = jax.lax.dot_general(xq, blk, (((1,), (1,)), ((), ())),
                            preferred_element_type=jnp.float32)
    ones_row = jnp.ones((1, xq.shape[1]), jnp.float32)
    k2 = jax.lax.dot_general(ones_row, blk * blk, (((1,), (1,)), ((), ())),
                             preferred_element_type=jnp.float32)
    col = jax.lax.broadcasted_iota(jnp.int32, (1, blk_rows), 1) + i * blk_rows
    t = jnp.where(col < n_keys, k2 - 2.0 * s, _INF)

    labf = labels_ref[...].astype(jnp.float32).reshape(1, blk_rows)

    vtiles, ltiles = [], []
    for p in range(n_panels):
        sl = slice(p * pw, (p + 1) * pw)
        bv, bl = _extract_top5(t[:, sl], labf[:, sl])
        vtiles.append(bv)
        ltiles.append(bl)
    pad = 128 - 8 * n_panels
    vtiles.append(jnp.full((q, pad), _INF, jnp.float32))
    ltiles.append(jnp.zeros((q, pad), jnp.float32))
    cval_ref[:, pl.ds(i * 128, 128)] = jnp.concatenate(vtiles, axis=1)
    clab_ref[:, pl.ds(i * 128, 128)] = jnp.concatenate(ltiles, axis=1)

    @pl.when(i == nb - 1)
    def _finish():
        fv, fl = _extract_top5(cval_ref[...], clab_ref[...])
        q2 = jnp.sum(xq * xq, axis=1, keepdims=True)
        d2 = jnp.maximum(fv + q2, 0.0)
        dist = jnp.sqrt(d2)
        w = 1.0 / jnp.maximum(dist, 1e-12)
        g_cols = []
        for k in range(8):
            same = fl[:, k:k + 1] == fl
            g_cols.append(jnp.sum(jnp.where(same, w, 0.0), axis=1,
                                  keepdims=True))
        g = jnp.concatenate(g_cols, axis=1)
        gm = jnp.max(g, axis=1, keepdims=True)
        pred = jnp.min(jnp.where(g == gm, fl, _BIG_LABEL), axis=1,
                       keepdims=True)
        cls = jax.lax.broadcasted_iota(jnp.int32, out_ref.shape, 1)
        out_ref[...] = (cls.astype(jnp.float32) == pred).astype(jnp.float32)


def kernel(x, data, labels, classes_mask):
    del classes_mask
    n_keys = data.shape[0]
    q = x.shape[0]
    n_classes = 1000
    blk_rows = 8192
    n_panels = 8
    nb = pl.cdiv(n_keys, blk_rows)

    body = functools.partial(_knn_body, n_keys, blk_rows, n_panels)
    return pl.pallas_call(
        body,
        grid=(nb,),
        in_specs=[
            pl.BlockSpec((q, data.shape[1]), lambda i: (0, 0)),
            pl.BlockSpec((blk_rows, data.shape[1]), lambda i: (i, 0)),
            pl.BlockSpec((blk_rows,), lambda i: (i,)),
        ],
        out_specs=pl.BlockSpec((q, n_classes), lambda i: (0, 0)),
        out_shape=jax.ShapeDtypeStruct((q, n_classes), jnp.float32),
        scratch_shapes=[
            pltpu.VMEM((q, nb * 128), jnp.float32),
            pltpu.VMEM((q, nb * 128), jnp.float32),
        ],
        compiler_params=pltpu.CompilerParams(
            dimension_semantics=("arbitrary",),
        ),
    )(x, data, labels)


# speedup vs baseline: 1.7480x; 1.0551x over previous
import functools

import jax
import jax.numpy as jnp
from jax.experimental import pallas as pl
from jax.experimental.pallas import tpu as pltpu

_INF = 1e30
_BIG_LABEL = 1e9
_K = 5
_LANES = 128


def _knn_body(n_keys, blk_rows, x_ref, data_ref, labels_ref, out_ref,
              accv_ref, accl_ref):
    i = pl.program_id(0)
    nb = pl.num_programs(0)
    q = x_ref.shape[0]
    n_slices = blk_rows // _LANES

    @pl.when(i == 0)
    def _init():
        accv_ref[...] = jnp.full(accv_ref.shape, -_INF, jnp.float32)
        accl_ref[...] = jnp.zeros(accl_ref.shape, jnp.float32)

    xq = x_ref[...]
    blk = data_ref[...]
    s = jax.lax.dot_general(xq, blk, (((1,), (1,)), ((), ())),
                            preferred_element_type=jnp.float32)
    ones_row = jnp.ones((1, xq.shape[1]), jnp.float32)
    h = 0.5 * jax.lax.dot_general(ones_row, blk * blk,
                                  (((1,), (1,)), ((), ())),
                                  preferred_element_type=jnp.float32)
    col = jax.lax.broadcasted_iota(jnp.int32, (1, blk_rows), 1) + i * blk_rows
    h = jnp.where(col < n_keys, h, _INF)

    labf = labels_ref[...].astype(jnp.float32).reshape(1, blk_rows)

    m = [accv_ref[:, pl.ds(k * _LANES, _LANES)] for k in range(_K)]
    l = [accl_ref[:, pl.ds(k * _LANES, _LANES)] for k in range(_K)]
    for sl_i in range(n_slices):
        sl = slice(sl_i * _LANES, (sl_i + 1) * _LANES)
        u = s[:, sl] - h[:, sl]
        lu = jnp.broadcast_to(labf[:, sl], (q, _LANES))
        for k in range(_K):
            cmp = u > m[k]
            nm = jnp.where(cmp, u, m[k])
            nl = jnp.where(cmp, lu, l[k])
            if k < _K - 1:
                nu = jnp.where(cmp, m[k], u)
                nlu = jnp.where(cmp, l[k], lu)
                u, lu = nu, nlu
            m[k] = nm
            l[k] = nl
    for k in range(_K):
        accv_ref[:, pl.ds(k * _LANES, _LANES)] = m[k]
        accl_ref[:, pl.ds(k * _LANES, _LANES)] = l[k]

    @pl.when(i == nb - 1)
    def _finish():
        cur = accv_ref[...]
        labcur = accl_ref[...]
        vals, labs = [], []
        for _ in range(_K):
            mx = jnp.max(cur, axis=1, keepdims=True)
            eq = cur == mx
            lb = jnp.min(jnp.where(eq, labcur, _BIG_LABEL), axis=1,
                         keepdims=True)
            vals.append(mx)
            labs.append(lb)
            cur = jnp.where(eq, -_INF, cur)
        fv = jnp.concatenate(
            vals + [jnp.full((q, 3), -_INF, jnp.float32)], axis=1)
        fl = jnp.concatenate(
            labs + [jnp.zeros((q, 3), jnp.float32)], axis=1)

        q2 = jnp.sum(xq * xq, axis=1, keepdims=True)
        d2 = jnp.maximum(q2 - 2.0 * fv, 0.0)
        dist = jnp.sqrt(d2)
        w = 1.0 / jnp.maximum(dist, 1e-12)
        w = jnp.where(fv <= -_INF, 0.0, w)
        g_cols = []
        for k in range(8):
            same = fl[:, k:k + 1] == fl
            g_cols.append(jnp.sum(jnp.where(same, w, 0.0), axis=1,
                                  keepdims=True))
        g = jnp.concatenate(g_cols, axis=1)
        gm = jnp.max(g, axis=1, keepdims=True)
        pred = jnp.min(jnp.where(g == gm, fl, _BIG_LABEL), axis=1,
                       keepdims=True)
        cls = jax.lax.broadcasted_iota(jnp.int32, out_ref.shape, 1)
        out_ref[...] = (cls.astype(jnp.float32) == pred).astype(jnp.float32)


def kernel(x, data, labels, classes_mask):
    del classes_mask
    n_keys = data.shape[0]
    q = x.shape[0]
    n_classes = 1000
    blk_rows = 8192
    nb = pl.cdiv(n_keys, blk_rows)

    body = functools.partial(_knn_body, n_keys, blk_rows)
    return pl.pallas_call(
        body,
        grid=(nb,),
        in_specs=[
            pl.BlockSpec((q, data.shape[1]), lambda i: (0, 0)),
            pl.BlockSpec((blk_rows, data.shape[1]), lambda i: (i, 0)),
            pl.BlockSpec((blk_rows,), lambda i: (i,)),
        ],
        out_specs=pl.BlockSpec((q, n_classes), lambda i: (0, 0)),
        out_shape=jax.ShapeDtypeStruct((q, n_classes), jnp.float32),
        scratch_shapes=[
            pltpu.VMEM((q, _K * _LANES), jnp.float32),
            pltpu.VMEM((q, _K * _LANES), jnp.float32),
        ],
        compiler_params=pltpu.CompilerParams(
            dimension_semantics=("arbitrary",),
        ),
    )(x, data, labels)
